# baseline (speedup 1.0000x reference)
import jax
import jax.numpy as jnp
from jax.experimental import pallas as pl
from jax.experimental.pallas import tpu as pltpu

B, A, C = 8, 20000, 80
CHUNK = 2000
CPAD = 2048
NCHUNK = A // CHUNK
NCAP = 1152
KTOP = 1000
KPAD = 1024
CONF_T = 0.5
IOU_T = 0.6
NEG = float("-inf")


def _reduce_kernel(s_ref, conf_ref, cls_ref):
    x = s_ref[0]
    maxv = jnp.max(x, axis=1)
    iot = jax.lax.broadcasted_iota(jnp.int32, (CHUNK, C), 1)
    am = jnp.min(jnp.where(x == maxv[:, None], iot, C), axis=1)
    conf_ref[0, 0, 0] = jnp.concatenate(
        [maxv, jnp.full((CPAD - CHUNK,), NEG, jnp.float32)])
    cls_ref[0, 0, 0] = jnp.concatenate(
        [am, jnp.zeros((CPAD - CHUNK,), jnp.int32)])


def _thresh_kernel(conf_ref, thr_ref):
    conf = conf_ref[...]

    def body(_, lohi):
        lo, hi = lohi
        mid = (lo + hi) // 2
        t_f = jax.lax.bitcast_convert_type(mid, jnp.float32)
        cnt = jnp.sum((conf > t_f).astype(jnp.int32), axis=1, keepdims=True)
        small = cnt <= (KTOP - 1)
        lo = jnp.where(small, lo, mid + 1)
        hi = jnp.where(small, mid, hi)
        return (lo, hi)

    lo0 = jnp.zeros((B, 1), jnp.int32)
    hi0 = jnp.full((B, 1), 0x3F800000, jnp.int32)
    lo, _ = jax.lax.fori_loop(0, 30, body, (lo0, hi0))
    thr_ref[...] = jax.lax.bitcast_convert_type(lo, jnp.float32).reshape(B, 1, 1)


def _compact_kernel(thr_ref, conf_ref, cls_ref, boxes_ref, out_ref, cnt_ref):
    a = pl.program_id(1)

    @pl.when(a == 0)
    def _():
        cnt_ref[0] = 0
        out_ref[...] = jnp.zeros_like(out_ref)

    base = cnt_ref[0]
    confr = conf_ref[0, 0]
    clsr = cls_ref[0, 0]
    sel = confr >= thr_ref[0]
    self32 = sel.astype(jnp.int32)

    x = self32
    liota = jax.lax.broadcasted_iota(jnp.int32, (1, CPAD), 1)
    sh = 1
    while sh < CPAD:
        x = x + jnp.where(liota >= sh, pltpu.roll(x, sh, axis=1), 0)
        sh *= 2
    pos = base + x - self32

    kiota = jax.lax.broadcasted_iota(jnp.int32, (NCAP, 1), 0)
    amat = ((pos == kiota) & sel).astype(jnp.float32)

    conf_c = jnp.maximum(jnp.transpose(confr), 0.0)
    cls_c = jnp.transpose(clsr).astype(jnp.float32)
    idx_c = (jax.lax.broadcasted_iota(jnp.int32, (CPAD, 1), 0)
             + a * CHUNK).astype(jnp.float32)
    ones_c = jnp.ones((CPAD, 1), jnp.float32)
    bx = boxes_ref[0]
    bx = jnp.concatenate([bx, jnp.zeros((CPAD - CHUNK, 4), jnp.float32)], axis=0)
    bmat = jnp.concatenate([conf_c, bx, cls_c, idx_c, ones_c], axis=1)

    out_ref[0] = out_ref[0] + jnp.dot(
        amat, bmat, preferred_element_type=jnp.float32,
        precision=jax.lax.Precision.HIGHEST)
    cnt_ref[0] = base + jnp.sum(self32)


def _nms_kernel(pay_ref, out_ref):
    pay = pay_ref[0]
    payt = jnp.transpose(pay)
    valid_c = pay[:, 7:8] > 0.5
    valid_r = payt[7:8] > 0.5
    conf_c = jnp.where(valid_c, pay[:, 0:1], NEG)
    conf_r = jnp.where(valid_r, payt[0:1], NEG)
    idx_c = jnp.where(valid_c, pay[:, 6:7], 1e9)
    idx_r = jnp.where(valid_r, payt[6:7], 1e9)

    SW = 128
    rstrips = []
    for s in range(NCAP // SW):
        cr = conf_r[:, s * SW:(s + 1) * SW]
        ir = idx_r[:, s * SW:(s + 1) * SW]
        gt = (conf_c > cr) | ((conf_c == cr) & (idx_c < ir))
        rstrips.append(jnp.sum(gt.astype(jnp.float32), axis=0, keepdims=True))
    rank_r = jnp.concatenate(rstrips, axis=1)
    rank_c = jnp.transpose(rank_r)

    x1c, y1c = pay[:, 1:2], pay[:, 2:3]
    x2c, y2c = pay[:, 3:4], pay[:, 4:5]
    area_c = (x2c - x1c) * (y2c - y1c)

    sstrips = []
    for s in range(NCAP // SW):
        x1r, y1r = payt[1:2, s * SW:(s + 1) * SW], payt[2:3, s * SW:(s + 1) * SW]
        x2r, y2r = payt[3:4, s * SW:(s + 1) * SW], payt[4:5, s * SW:(s + 1) * SW]
        area_r = (x2r - x1r) * (y2r - y1r)
        xx1 = jnp.maximum(x1c, x1r)
        yy1 = jnp.maximum(y1c, y1r)
        xx2 = jnp.minimum(x2c, x2r)
        yy2 = jnp.minimum(y2c, y2r)
        w = jnp.maximum(xx2 - xx1, 0.0)
        h = jnp.maximum(yy2 - yy1, 0.0)
        inter = w * h
        iou = inter / jnp.maximum(area_c + area_r - inter, 1e-9)
        mask = rank_c < rank_r[:, s * SW:(s + 1) * SW]
        sstrips.append(jnp.max(jnp.where(mask, iou, 0.0), axis=0, keepdims=True))
    supp = jnp.concatenate(sstrips, axis=1)

    keep_r = (supp <= IOU_T) & (conf_r > CONF_T) & (rank_r < KTOP)
    keep_c = jnp.transpose(keep_r)

    bcol = jnp.full((NCAP, 1), pl.program_id(0).astype(jnp.float32))
    keep_f = keep_c.astype(jnp.float32)
    rows = jnp.concatenate(
        [pay[:, 0:5] * keep_f, bcol, pay[:, 5:6] * keep_f,
         jnp.zeros((NCAP, 1), jnp.float32)],
        axis=1)

    kio = jax.lax.broadcasted_iota(jnp.int32, (KPAD, 1), 0).astype(jnp.float32)
    sc = (rank_r == kio).astype(jnp.float32)
    out_ref[0] = jnp.dot(sc, rows, preferred_element_type=jnp.float32,
                         precision=jax.lax.Precision.HIGHEST)


def _stage_reduce(scores):
    return pl.pallas_call(
        _reduce_kernel,
        grid=(B, NCHUNK),
        in_specs=[pl.BlockSpec((1, CHUNK, C), lambda b, a: (b, a, 0))],
        out_specs=[pl.BlockSpec((1, 1, 1, CPAD), lambda b, a: (b, a, 0, 0)),
                   pl.BlockSpec((1, 1, 1, CPAD), lambda b, a: (b, a, 0, 0))],
        out_shape=[jax.ShapeDtypeStruct((B, NCHUNK, 1, CPAD), jnp.float32),
                   jax.ShapeDtypeStruct((B, NCHUNK, 1, CPAD), jnp.int32)],
    )(scores)


def _stage_thresh(conf):
    return pl.pallas_call(
        _thresh_kernel,
        out_shape=jax.ShapeDtypeStruct((B, 1, 1), jnp.float32),
    )(conf.reshape(B, NCHUNK * CPAD))


def _stage_compact(thr, conf, cls, boxes):
    return pl.pallas_call(
        _compact_kernel,
        grid=(B, NCHUNK),
        in_specs=[pl.BlockSpec((1, 1, 1), lambda b, a: (b, 0, 0)),
                  pl.BlockSpec((1, 1, 1, CPAD), lambda b, a: (b, a, 0, 0)),
                  pl.BlockSpec((1, 1, 1, CPAD), lambda b, a: (b, a, 0, 0)),
                  pl.BlockSpec((1, CHUNK, 4), lambda b, a: (b, a, 0))],
        out_specs=pl.BlockSpec((1, NCAP, 8), lambda b, a: (b, 0, 0)),
        out_shape=jax.ShapeDtypeStruct((B, NCAP, 8), jnp.float32),
        scratch_shapes=[pltpu.SMEM((1,), jnp.int32)],
    )(thr, conf, cls, boxes)


def _stage_nms(pay):
    return pl.pallas_call(
        _nms_kernel,
        grid=(B,),
        in_specs=[pl.BlockSpec((1, NCAP, 8), lambda b: (b, 0, 0))],
        out_specs=pl.BlockSpec((1, KPAD, 8), lambda b: (b, 0, 0)),
        out_shape=jax.ShapeDtypeStruct((B, KPAD, 8), jnp.float32),
    )(pay)


def kernel(boxes, scores):
    conf, cls = _stage_reduce(scores)
    thr = _stage_thresh(conf)
    pay = _stage_compact(thr, conf, cls, boxes)
    res = _stage_nms(pay)
    return res[:, :KTOP, :7]
